# baseline (speedup 1.0000x reference)
import functools

import jax
import jax.numpy as jnp
from jax import lax
from jax.experimental import pallas as pl
from jax.experimental.pallas import tpu as pltpu
from jax.experimental.pallas import tpu_sc as plsc

K = 16
R_BLK = 256
M_BLK = 128
N_SEG = 32


def _make_knn_body(n_real, n_pad, r_blk, k, n_seg):
    seg_w = n_pad // n_seg

    def body(xb_ref, xT_ref, w1d_ref, w1b_ref, b1_ref, idx_ref, p_ref, q_ref,
             dist_ref, cval_ref, cidx_ref, done_ref):
        i = pl.program_id(0)
        xb = xb_ref[...]
        p_ref[...] = jnp.dot(xb, w1d_ref[...],
                             preferred_element_type=jnp.float32) + b1_ref[...]
        q_ref[...] = jnp.dot(xb, w1b_ref[...],
                             preferred_element_type=jnp.float32)

        xT = xT_ref[...]
        s = lax.dot_general(xb, xT, (((1,), (0,)), ((), ())),
                            preferred_element_type=jnp.float32)
        row_sq = jnp.sum(xb * xb, axis=1, keepdims=True)
        col_sq = jnp.sum(xT * xT, axis=0, keepdims=True)
        dist = row_sq + col_sq - 2.0 * s

        colid = lax.broadcasted_iota(jnp.int32, (1, n_pad), 1)
        rowid = i * r_blk + lax.broadcasted_iota(jnp.int32, (r_blk, 1), 0)
        dist = jnp.where((colid == rowid) | (colid >= n_real), jnp.inf, dist)
        dist_ref[...] = dist.reshape(r_blk, n_seg, seg_w)

        cval_ref[...] = jnp.full((k, r_blk, n_seg), jnp.inf, jnp.float32)
        cidx_ref[...] = jnp.zeros((k, r_blk, n_seg), jnp.int32)
        done_ref[0] = 0

        colid3 = (lax.broadcasted_iota(jnp.int32, (1, n_seg, seg_w), 1) * seg_w
                  + lax.broadcasted_iota(jnp.int32, (1, n_seg, seg_w), 2))

        def round_body(t, carry):
            @pl.when(done_ref[0] == 0)
            def _():
                d3 = dist_ref[...]
                m_seg = jnp.min(d3, axis=2)
                floor = jnp.min(m_seg, axis=1, keepdims=True)
                sofar = jnp.concatenate(
                    [cval_ref[t2] for t2 in range(k)], axis=1)
                have = jnp.sum((sofar < floor).astype(jnp.int32), axis=1)
                all_ok = jnp.min(have) >= k

                @pl.when(all_ok)
                def _():
                    done_ref[0] = 1

                @pl.when(jnp.logical_not(all_ok))
                def _():
                    eq = d3 == m_seg[:, :, None]
                    am = jnp.min(jnp.where(eq, colid3, n_pad), axis=2)
                    dist_ref[...] = jnp.where(colid3 == am[:, :, None],
                                              jnp.inf, d3)
                    cval_ref[t] = m_seg
                    cidx_ref[t] = am
            return carry

        lax.fori_loop(0, k, round_body, 0)

        vals = jnp.concatenate([cval_ref[t2] for t2 in range(k)], axis=1)
        gidx = jnp.concatenate([cidx_ref[t2] for t2 in range(k)], axis=1)

        idxs = []
        for j in range(k):
            m = jnp.min(vals, axis=1, keepdims=True)
            am = jnp.min(jnp.where(vals == m, gidx, n_pad),
                         axis=1, keepdims=True)
            vals = jnp.where(gidx == am, jnp.inf, vals)
            am = jnp.where(rowid >= n_real,
                           (rowid * k + j) % n_real, am)
            idxs.append(am)
        idx_ref[...] = jnp.concatenate(idxs, axis=1)

    return body


def _make_mlp2_body(k):
    def body(qg_ref, p_ref, w2_ref, b2_ref, o_ref):
        p = p_ref[...]
        w2 = w2_ref[...]
        acc = None
        for j in range(k):
            h = jnp.maximum(p + qg_ref[j], 0.0)
            e = jnp.dot(h, w2, preferred_element_type=jnp.float32)
            acc = e if acc is None else jnp.maximum(acc, e)
        o_ref[...] = acc + b2_ref[...]

    return body


def _sc_gather(table, idx_flat, d):
    b_total = idx_flat.shape[0]
    info = plsc.get_sparse_core_info()
    nw = info.num_cores * info.num_subcores
    b_per_w = b_total // nw
    ch = 128
    n_ch = b_per_w // ch
    mesh = plsc.VectorSubcoreMesh(core_axis_name="c", subcore_axis_name="s")

    @functools.partial(
        pl.kernel, mesh=mesh,
        out_type=jax.ShapeDtypeStruct((b_total, d), jnp.float32),
        scratch_types=[
            pltpu.VMEM((ch,), jnp.int32),
            pltpu.VMEM((ch, d), jnp.float32),
            pltpu.SemaphoreType.DMA,
        ],
    )
    def gk(table_hbm, idx_hbm, out_hbm, idx_v, rows_v, sem):
        wid = lax.axis_index("s") * info.num_cores + lax.axis_index("c")
        base = wid * b_per_w

        def step(c, carry):
            off = base + c * ch
            pltpu.sync_copy(idx_hbm.at[pl.ds(off, ch)], idx_v)
            pltpu.async_copy(table_hbm.at[idx_v], rows_v, sem).wait()
            pltpu.sync_copy(rows_v, out_hbm.at[pl.ds(off, ch)])
            return carry

        lax.fori_loop(0, n_ch, step, 0)

    return gk(table, idx_flat)


def kernel(x, W1, b1, W2, b2):
    n_real, d = x.shape
    h = W1.shape[1]
    out_dim = W2.shape[1]
    n_pad = ((n_real + R_BLK - 1) // R_BLK) * R_BLK

    xp = jnp.pad(x, ((0, n_pad - n_real), (0, 0)))
    xT = xp.T
    w1d = W1[:d] - W1[d:]
    w1b = W1[d:]
    b1r = b1.reshape(1, h)

    n_blocks = n_pad // R_BLK
    idx, p, q = pl.pallas_call(
        _make_knn_body(n_real, n_pad, R_BLK, K, N_SEG),
        grid=(n_blocks,),
        scratch_shapes=[
            pltpu.VMEM((R_BLK, N_SEG, n_pad // N_SEG), jnp.float32),
            pltpu.VMEM((K, R_BLK, N_SEG), jnp.float32),
            pltpu.VMEM((K, R_BLK, N_SEG), jnp.int32),
            pltpu.SMEM((1,), jnp.int32),
        ],
        in_specs=[
            pl.BlockSpec((R_BLK, d), lambda i: (i, 0)),
            pl.BlockSpec((d, n_pad), lambda i: (0, 0)),
            pl.BlockSpec((d, h), lambda i: (0, 0)),
            pl.BlockSpec((d, h), lambda i: (0, 0)),
            pl.BlockSpec((1, h), lambda i: (0, 0)),
        ],
        out_specs=[
            pl.BlockSpec((R_BLK, K), lambda i: (i, 0)),
            pl.BlockSpec((R_BLK, h), lambda i: (i, 0)),
            pl.BlockSpec((R_BLK, h), lambda i: (i, 0)),
        ],
        out_shape=[
            jax.ShapeDtypeStruct((n_pad, K), jnp.int32),
            jax.ShapeDtypeStruct((n_pad, h), jnp.float32),
            jax.ShapeDtypeStruct((n_pad, h), jnp.float32),
        ],
    )(xp, xT, w1d, w1b, b1r)

    idx_flat = idx.T.reshape(-1)
    qg = _sc_gather(q, idx_flat, h)
    qg = qg.reshape(K, n_pad, h)

    n_mblocks = n_pad // M_BLK
    out = pl.pallas_call(
        _make_mlp2_body(K),
        grid=(n_mblocks,),
        in_specs=[
            pl.BlockSpec((K, M_BLK, h), lambda i: (0, i, 0)),
            pl.BlockSpec((M_BLK, h), lambda i: (i, 0)),
            pl.BlockSpec((h, out_dim), lambda i: (0, 0)),
            pl.BlockSpec((1, out_dim), lambda i: (0, 0)),
        ],
        out_specs=pl.BlockSpec((M_BLK, out_dim), lambda i: (i, 0)),
        out_shape=jax.ShapeDtypeStruct((n_pad, out_dim), jnp.float32),
    )(qg, p, W2, b2.reshape(1, out_dim))

    return out[:n_real]


# speedup vs baseline: 1.0421x; 1.0421x over previous
import functools

import jax
import jax.numpy as jnp
from jax import lax
from jax.experimental import pallas as pl
from jax.experimental.pallas import tpu as pltpu
from jax.experimental.pallas import tpu_sc as plsc

K = 16
R_BLK = 256
M_BLK = 128
N_SEG = 40


def _make_knn_body(n_real, n_pad, r_blk, k, n_seg):
    seg_w = n_pad // n_seg

    def body(xb_ref, xT_ref, w1d_ref, w1b_ref, b1_ref, idx_ref, p_ref, q_ref,
             dist_ref, cval_ref, cidx_ref, done_ref):
        i = pl.program_id(0)
        xb = xb_ref[...]
        p_ref[...] = jnp.dot(xb, w1d_ref[...],
                             preferred_element_type=jnp.float32) + b1_ref[...]
        q_ref[...] = jnp.dot(xb, w1b_ref[...],
                             preferred_element_type=jnp.float32)

        xT = xT_ref[...]
        s = lax.dot_general(xb, xT, (((1,), (0,)), ((), ())),
                            preferred_element_type=jnp.float32)
        row_sq = jnp.sum(xb * xb, axis=1, keepdims=True)
        col_sq = jnp.sum(xT * xT, axis=0, keepdims=True)
        dist = row_sq + col_sq - 2.0 * s

        colid = lax.broadcasted_iota(jnp.int32, (1, n_pad), 1)
        rowid = i * r_blk + lax.broadcasted_iota(jnp.int32, (r_blk, 1), 0)
        dist = jnp.where((colid == rowid) | (colid >= n_real), jnp.inf, dist)
        dist_ref[...] = dist.reshape(r_blk, n_seg, seg_w)

        cval_ref[...] = jnp.full((k, r_blk, n_seg), jnp.inf, jnp.float32)
        cidx_ref[...] = jnp.zeros((k, r_blk, n_seg), jnp.int32)
        done_ref[0] = 0

        colid3 = (lax.broadcasted_iota(jnp.int32, (1, n_seg, seg_w), 1) * seg_w
                  + lax.broadcasted_iota(jnp.int32, (1, n_seg, seg_w), 2))

        def round_body(t, carry):
            @pl.when(done_ref[0] == 0)
            def _():
                d3 = dist_ref[...]
                m_seg = jnp.min(d3, axis=2)
                floor = jnp.min(m_seg, axis=1, keepdims=True)
                sofar = jnp.concatenate(
                    [cval_ref[t2] for t2 in range(k)], axis=1)
                have = jnp.sum((sofar < floor).astype(jnp.int32), axis=1)
                all_ok = jnp.min(have) >= k

                @pl.when(all_ok)
                def _():
                    done_ref[0] = 1

                @pl.when(jnp.logical_not(all_ok))
                def _():
                    eq = d3 == m_seg[:, :, None]
                    am = jnp.min(jnp.where(eq, colid3, n_pad), axis=2)
                    dist_ref[...] = jnp.where(colid3 == am[:, :, None],
                                              jnp.inf, d3)
                    cval_ref[t] = m_seg
                    cidx_ref[t] = am
            return carry

        lax.fori_loop(0, k, round_body, 0)

        vals = jnp.concatenate([cval_ref[t2] for t2 in range(k)], axis=1)
        gidx = jnp.concatenate([cidx_ref[t2] for t2 in range(k)], axis=1)

        idxs = []
        for j in range(k):
            m = jnp.min(vals, axis=1, keepdims=True)
            am = jnp.min(jnp.where(vals == m, gidx, n_pad),
                         axis=1, keepdims=True)
            vals = jnp.where(gidx == am, jnp.inf, vals)
            am = jnp.where(rowid >= n_real,
                           (rowid * k + j) % n_real, am)
            idxs.append(am)
        idx_ref[...] = jnp.concatenate(idxs, axis=1)

    return body


def _make_mlp2_body(k):
    def body(qg_ref, p_ref, w2_ref, b2_ref, o_ref):
        p = p_ref[...]
        w2 = w2_ref[...]
        acc = None
        for j in range(k):
            h = jnp.maximum(p + qg_ref[j], 0.0)
            e = jnp.dot(h, w2, preferred_element_type=jnp.float32)
            acc = e if acc is None else jnp.maximum(acc, e)
        o_ref[...] = acc + b2_ref[...]

    return body


def _sc_gather(table, idx_flat, d):
    b_total = idx_flat.shape[0]
    info = plsc.get_sparse_core_info()
    nw = info.num_cores * info.num_subcores
    b_per_w = b_total // nw
    ch = 128
    n_ch = b_per_w // ch
    mesh = plsc.VectorSubcoreMesh(core_axis_name="c", subcore_axis_name="s")

    @functools.partial(
        pl.kernel, mesh=mesh,
        out_type=jax.ShapeDtypeStruct((b_total, d), jnp.float32),
        scratch_types=[
            pltpu.VMEM((ch,), jnp.int32),
            pltpu.VMEM((ch, d), jnp.float32),
            pltpu.SemaphoreType.DMA,
        ],
    )
    def gk(table_hbm, idx_hbm, out_hbm, idx_v, rows_v, sem):
        wid = lax.axis_index("s") * info.num_cores + lax.axis_index("c")
        base = wid * b_per_w

        def step(c, carry):
            off = base + c * ch
            pltpu.sync_copy(idx_hbm.at[pl.ds(off, ch)], idx_v)
            pltpu.async_copy(table_hbm.at[idx_v], rows_v, sem).wait()
            pltpu.sync_copy(rows_v, out_hbm.at[pl.ds(off, ch)])
            return carry

        lax.fori_loop(0, n_ch, step, 0)

    return gk(table, idx_flat)


def kernel(x, W1, b1, W2, b2):
    n_real, d = x.shape
    h = W1.shape[1]
    out_dim = W2.shape[1]
    n_pad = ((n_real + R_BLK - 1) // R_BLK) * R_BLK

    xp = jnp.pad(x, ((0, n_pad - n_real), (0, 0)))
    xT = xp.T
    w1d = W1[:d] - W1[d:]
    w1b = W1[d:]
    b1r = b1.reshape(1, h)

    n_blocks = n_pad // R_BLK
    idx, p, q = pl.pallas_call(
        _make_knn_body(n_real, n_pad, R_BLK, K, N_SEG),
        grid=(n_blocks,),
        scratch_shapes=[
            pltpu.VMEM((R_BLK, N_SEG, n_pad // N_SEG), jnp.float32),
            pltpu.VMEM((K, R_BLK, N_SEG), jnp.float32),
            pltpu.VMEM((K, R_BLK, N_SEG), jnp.int32),
            pltpu.SMEM((1,), jnp.int32),
        ],
        in_specs=[
            pl.BlockSpec((R_BLK, d), lambda i: (i, 0)),
            pl.BlockSpec((d, n_pad), lambda i: (0, 0)),
            pl.BlockSpec((d, h), lambda i: (0, 0)),
            pl.BlockSpec((d, h), lambda i: (0, 0)),
            pl.BlockSpec((1, h), lambda i: (0, 0)),
        ],
        out_specs=[
            pl.BlockSpec((R_BLK, K), lambda i: (i, 0)),
            pl.BlockSpec((R_BLK, h), lambda i: (i, 0)),
            pl.BlockSpec((R_BLK, h), lambda i: (i, 0)),
        ],
        out_shape=[
            jax.ShapeDtypeStruct((n_pad, K), jnp.int32),
            jax.ShapeDtypeStruct((n_pad, h), jnp.float32),
            jax.ShapeDtypeStruct((n_pad, h), jnp.float32),
        ],
    )(xp, xT, w1d, w1b, b1r)

    idx_flat = idx.T.reshape(-1)
    qg = _sc_gather(q, idx_flat, h)
    qg = qg.reshape(K, n_pad, h)

    n_mblocks = n_pad // M_BLK
    out = pl.pallas_call(
        _make_mlp2_body(K),
        grid=(n_mblocks,),
        in_specs=[
            pl.BlockSpec((K, M_BLK, h), lambda i: (0, i, 0)),
            pl.BlockSpec((M_BLK, h), lambda i: (i, 0)),
            pl.BlockSpec((h, out_dim), lambda i: (0, 0)),
            pl.BlockSpec((1, out_dim), lambda i: (0, 0)),
        ],
        out_specs=pl.BlockSpec((M_BLK, out_dim), lambda i: (i, 0)),
        out_shape=jax.ShapeDtypeStruct((n_pad, out_dim), jnp.float32),
    )(qg, p, W2, b2.reshape(1, out_dim))

    return out[:n_real]


# speedup vs baseline: 1.1154x; 1.0703x over previous
import functools

import jax
import jax.numpy as jnp
from jax import lax
from jax.experimental import pallas as pl
from jax.experimental.pallas import tpu as pltpu
from jax.experimental.pallas import tpu_sc as plsc

K = 16
R_BLK = 256
M_BLK = 128
N_SEG = 16


def _make_knn_body(n_real, n_pad, r_blk, k, n_seg):
    seg_w = n_pad // n_seg

    def body(xb_ref, xT_ref, w1d_ref, w1b_ref, b1_ref, idx_ref, p_ref, q_ref,
             dist_ref, cval_ref, cidx_ref, done_ref):
        i = pl.program_id(0)
        xb = xb_ref[...]
        p_ref[...] = jnp.dot(xb, w1d_ref[...],
                             preferred_element_type=jnp.float32) + b1_ref[...]
        q_ref[...] = jnp.dot(xb, w1b_ref[...],
                             preferred_element_type=jnp.float32)

        xT = xT_ref[...]
        s = lax.dot_general(xb, xT, (((1,), (0,)), ((), ())),
                            preferred_element_type=jnp.float32)
        row_sq = jnp.sum(xb * xb, axis=1, keepdims=True)
        col_sq = jnp.sum(xT * xT, axis=0, keepdims=True)
        dist = row_sq + col_sq - 2.0 * s

        colid = lax.broadcasted_iota(jnp.int32, (1, n_pad), 1)
        rowid = i * r_blk + lax.broadcasted_iota(jnp.int32, (r_blk, 1), 0)
        dist = jnp.where((colid == rowid) | (colid >= n_real), jnp.inf, dist)
        dist_ref[...] = dist

        cval_ref[...] = jnp.full((k, r_blk, n_seg), jnp.inf, jnp.float32)
        cidx_ref[...] = jnp.zeros((k, r_blk, n_seg), jnp.int32)
        done_ref[0] = 0

        colid_seg = lax.broadcasted_iota(jnp.int32, (1, seg_w), 1)

        def round_body(t, carry):
            @pl.when(done_ref[0] == 0)
            def _():
                segs = [dist_ref[:, s * seg_w:(s + 1) * seg_w]
                        for s in range(n_seg)]
                mins = [jnp.min(d, axis=1, keepdims=True) for d in segs]
                m_seg = jnp.concatenate(mins, axis=1)
                floor = jnp.min(m_seg, axis=1, keepdims=True)
                sofar = jnp.concatenate(
                    [cval_ref[t2] for t2 in range(k)], axis=1)
                have = jnp.sum((sofar < floor).astype(jnp.int32), axis=1)
                all_ok = jnp.min(have) >= k

                @pl.when(all_ok)
                def _():
                    done_ref[0] = 1

                @pl.when(jnp.logical_not(all_ok))
                def _():
                    ams = []
                    for s in range(n_seg):
                        am_s = jnp.min(
                            jnp.where(segs[s] == mins[s],
                                      colid_seg, seg_w),
                            axis=1, keepdims=True)
                        dist_ref[:, s * seg_w:(s + 1) * seg_w] = jnp.where(
                            colid_seg == am_s, jnp.inf, segs[s])
                        ams.append(am_s + s * seg_w)
                    cval_ref[t] = m_seg
                    cidx_ref[t] = jnp.concatenate(ams, axis=1)
            return carry

        lax.fori_loop(0, k, round_body, 0)

        vals = jnp.concatenate([cval_ref[t2] for t2 in range(k)], axis=1)
        gidx = jnp.concatenate([cidx_ref[t2] for t2 in range(k)], axis=1)

        idxs = []
        for j in range(k):
            m = jnp.min(vals, axis=1, keepdims=True)
            am = jnp.min(jnp.where(vals == m, gidx, n_pad),
                         axis=1, keepdims=True)
            vals = jnp.where(gidx == am, jnp.inf, vals)
            am = jnp.where(rowid >= n_real,
                           (rowid * k + j) % n_real, am)
            idxs.append(am)
        idx_ref[...] = jnp.concatenate(idxs, axis=1)

    return body


def _make_mlp2_body(k):
    def body(qg_ref, p_ref, w2_ref, b2_ref, o_ref):
        p = p_ref[...]
        w2 = w2_ref[...]
        acc = None
        for j in range(k):
            h = jnp.maximum(p + qg_ref[j], 0.0)
            e = jnp.dot(h, w2, preferred_element_type=jnp.float32)
            acc = e if acc is None else jnp.maximum(acc, e)
        o_ref[...] = acc + b2_ref[...]

    return body


def _sc_gather(table, idx_flat, d):
    b_total = idx_flat.shape[0]
    info = plsc.get_sparse_core_info()
    nw = info.num_cores * info.num_subcores
    b_per_w = b_total // nw
    ch = 128
    n_ch = b_per_w // ch
    mesh = plsc.VectorSubcoreMesh(core_axis_name="c", subcore_axis_name="s")

    @functools.partial(
        pl.kernel, mesh=mesh,
        out_type=jax.ShapeDtypeStruct((b_total, d), jnp.float32),
        scratch_types=[
            pltpu.VMEM((ch,), jnp.int32),
            pltpu.VMEM((ch, d), jnp.float32),
            pltpu.SemaphoreType.DMA,
        ],
    )
    def gk(table_hbm, idx_hbm, out_hbm, idx_v, rows_v, sem):
        wid = lax.axis_index("s") * info.num_cores + lax.axis_index("c")
        base = wid * b_per_w

        def step(c, carry):
            off = base + c * ch
            pltpu.sync_copy(idx_hbm.at[pl.ds(off, ch)], idx_v)
            pltpu.async_copy(table_hbm.at[idx_v], rows_v, sem).wait()
            pltpu.sync_copy(rows_v, out_hbm.at[pl.ds(off, ch)])
            return carry

        lax.fori_loop(0, n_ch, step, 0)

    return gk(table, idx_flat)


def kernel(x, W1, b1, W2, b2):
    n_real, d = x.shape
    h = W1.shape[1]
    out_dim = W2.shape[1]
    n_pad = ((n_real + R_BLK - 1) // R_BLK) * R_BLK

    xp = jnp.pad(x, ((0, n_pad - n_real), (0, 0)))
    xT = xp.T
    w1d = W1[:d] - W1[d:]
    w1b = W1[d:]
    b1r = b1.reshape(1, h)

    n_blocks = n_pad // R_BLK
    idx, p, q = pl.pallas_call(
        _make_knn_body(n_real, n_pad, R_BLK, K, N_SEG),
        grid=(n_blocks,),
        scratch_shapes=[
            pltpu.VMEM((R_BLK, n_pad), jnp.float32),
            pltpu.VMEM((K, R_BLK, N_SEG), jnp.float32),
            pltpu.VMEM((K, R_BLK, N_SEG), jnp.int32),
            pltpu.SMEM((1,), jnp.int32),
        ],
        in_specs=[
            pl.BlockSpec((R_BLK, d), lambda i: (i, 0)),
            pl.BlockSpec((d, n_pad), lambda i: (0, 0)),
            pl.BlockSpec((d, h), lambda i: (0, 0)),
            pl.BlockSpec((d, h), lambda i: (0, 0)),
            pl.BlockSpec((1, h), lambda i: (0, 0)),
        ],
        out_specs=[
            pl.BlockSpec((R_BLK, K), lambda i: (i, 0)),
            pl.BlockSpec((R_BLK, h), lambda i: (i, 0)),
            pl.BlockSpec((R_BLK, h), lambda i: (i, 0)),
        ],
        out_shape=[
            jax.ShapeDtypeStruct((n_pad, K), jnp.int32),
            jax.ShapeDtypeStruct((n_pad, h), jnp.float32),
            jax.ShapeDtypeStruct((n_pad, h), jnp.float32),
        ],
    )(xp, xT, w1d, w1b, b1r)

    idx_flat = idx.T.reshape(-1)
    qg = _sc_gather(q, idx_flat, h)
    qg = qg.reshape(K, n_pad, h)

    n_mblocks = n_pad // M_BLK
    out = pl.pallas_call(
        _make_mlp2_body(K),
        grid=(n_mblocks,),
        in_specs=[
            pl.BlockSpec((K, M_BLK, h), lambda i: (0, i, 0)),
            pl.BlockSpec((M_BLK, h), lambda i: (i, 0)),
            pl.BlockSpec((h, out_dim), lambda i: (0, 0)),
            pl.BlockSpec((1, out_dim), lambda i: (0, 0)),
        ],
        out_specs=pl.BlockSpec((M_BLK, out_dim), lambda i: (i, 0)),
        out_shape=jax.ShapeDtypeStruct((n_pad, out_dim), jnp.float32),
    )(qg, p, W2, b2.reshape(1, out_dim))

    return out[:n_real]


# speedup vs baseline: 1.1693x; 1.0483x over previous
import functools

import jax
import jax.numpy as jnp
from jax import lax
from jax.experimental import pallas as pl
from jax.experimental.pallas import tpu as pltpu
from jax.experimental.pallas import tpu_sc as plsc

K = 16
R_BLK = 256
M_BLK = 128
N_SEG = 16
SPLIT = 2


def _make_knn_body(n_real, n_pad, r_blk, k, n_seg, row_blk_off):
    seg_w = n_pad // n_seg

    def body(xb_ref, xT_ref, idx_ref,
             dist_ref, cval_ref, cidx_ref, done_ref):
        i = pl.program_id(0) + row_blk_off
        xb = xb_ref[...]
        xT = xT_ref[...]
        s = lax.dot_general(xb, xT, (((1,), (0,)), ((), ())),
                            preferred_element_type=jnp.float32)
        row_sq = jnp.sum(xb * xb, axis=1, keepdims=True)
        col_sq = jnp.sum(xT * xT, axis=0, keepdims=True)
        dist = row_sq + col_sq - 2.0 * s

        colid = lax.broadcasted_iota(jnp.int32, (1, n_pad), 1)
        rowid = i * r_blk + lax.broadcasted_iota(jnp.int32, (r_blk, 1), 0)
        dist = jnp.where((colid == rowid) | (colid >= n_real), jnp.inf, dist)
        dist_ref[...] = dist

        cval_ref[...] = jnp.full((k, r_blk, n_seg), jnp.inf, jnp.float32)
        cidx_ref[...] = jnp.zeros((k, r_blk, n_seg), jnp.int32)
        done_ref[0] = 0

        colid_seg = lax.broadcasted_iota(jnp.int32, (1, seg_w), 1)

        def round_body(t, carry):
            @pl.when(done_ref[0] == 0)
            def _():
                segs = [dist_ref[:, s * seg_w:(s + 1) * seg_w]
                        for s in range(n_seg)]
                mins = [jnp.min(d, axis=1, keepdims=True) for d in segs]
                m_seg = jnp.concatenate(mins, axis=1)
                floor = jnp.min(m_seg, axis=1, keepdims=True)
                sofar = jnp.concatenate(
                    [cval_ref[t2] for t2 in range(k)], axis=1)
                have = jnp.sum((sofar < floor).astype(jnp.int32), axis=1)
                all_ok = jnp.min(have) >= k

                @pl.when(all_ok)
                def _():
                    done_ref[0] = 1

                @pl.when(jnp.logical_not(all_ok))
                def _():
                    ams = []
                    for s in range(n_seg):
                        am_s = jnp.min(
                            jnp.where(segs[s] == mins[s],
                                      colid_seg, seg_w),
                            axis=1, keepdims=True)
                        dist_ref[:, s * seg_w:(s + 1) * seg_w] = jnp.where(
                            colid_seg == am_s, jnp.inf, segs[s])
                        ams.append(am_s + s * seg_w)
                    cval_ref[t] = m_seg
                    cidx_ref[t] = jnp.concatenate(ams, axis=1)
            return carry

        lax.fori_loop(0, k, round_body, 0)

        vals = jnp.concatenate([cval_ref[t2] for t2 in range(k)], axis=1)
        gidx = jnp.concatenate([cidx_ref[t2] for t2 in range(k)], axis=1)

        idxs = []
        for j in range(k):
            m = jnp.min(vals, axis=1, keepdims=True)
            am = jnp.min(jnp.where(vals == m, gidx, n_pad),
                         axis=1, keepdims=True)
            vals = jnp.where(gidx == am, jnp.inf, vals)
            am = jnp.where(rowid >= n_real,
                           (rowid * k + j) % n_real, am)
            idxs.append(am)
        idx_ref[...] = jnp.concatenate(idxs, axis=1)

    return body


def _make_pq_body():
    def body(xb_ref, w1d_ref, w1b_ref, b1_ref, p_ref, q_ref):
        xb = xb_ref[...]
        p_ref[...] = jnp.dot(xb, w1d_ref[...],
                             preferred_element_type=jnp.float32) + b1_ref[...]
        q_ref[...] = jnp.dot(xb, w1b_ref[...],
                             preferred_element_type=jnp.float32)

    return body


def _make_mlp2_body(k):
    def body(qg_ref, p_ref, w2_ref, b2_ref, o_ref):
        p = p_ref[...]
        w2 = w2_ref[...]
        acc = None
        for j in range(k):
            h = jnp.maximum(p + qg_ref[j], 0.0)
            e = jnp.dot(h, w2, preferred_element_type=jnp.float32)
            acc = e if acc is None else jnp.maximum(acc, e)
        o_ref[...] = acc + b2_ref[...]

    return body


def _sc_gather(table, idx_flat, d):
    b_total = idx_flat.shape[0]
    info = plsc.get_sparse_core_info()
    nw = info.num_cores * info.num_subcores
    b_per_w = b_total // nw
    ch = 128
    n_ch = b_per_w // ch
    mesh = plsc.VectorSubcoreMesh(core_axis_name="c", subcore_axis_name="s")

    @functools.partial(
        pl.kernel, mesh=mesh,
        out_type=jax.ShapeDtypeStruct((b_total, d), jnp.float32),
        scratch_types=[
            pltpu.VMEM((ch,), jnp.int32),
            pltpu.VMEM((ch, d), jnp.float32),
            pltpu.SemaphoreType.DMA,
        ],
    )
    def gk(table_hbm, idx_hbm, out_hbm, idx_v, rows_v, sem):
        wid = lax.axis_index("s") * info.num_cores + lax.axis_index("c")
        base = wid * b_per_w

        def step(c, carry):
            off = base + c * ch
            pltpu.sync_copy(idx_hbm.at[pl.ds(off, ch)], idx_v)
            pltpu.async_copy(table_hbm.at[idx_v], rows_v, sem).wait()
            pltpu.sync_copy(rows_v, out_hbm.at[pl.ds(off, ch)])
            return carry

        lax.fori_loop(0, n_ch, step, 0)

    return gk(table, idx_flat)


def kernel(x, W1, b1, W2, b2):
    n_real, d = x.shape
    h = W1.shape[1]
    out_dim = W2.shape[1]
    n_pad = ((n_real + R_BLK - 1) // R_BLK) * R_BLK

    xp = jnp.pad(x, ((0, n_pad - n_real), (0, 0)))
    xT = xp.T
    w1d = W1[:d] - W1[d:]
    w1b = W1[d:]
    b1r = b1.reshape(1, h)

    p, q = pl.pallas_call(
        _make_pq_body(),
        grid=(n_pad // R_BLK,),
        in_specs=[
            pl.BlockSpec((R_BLK, d), lambda i: (i, 0)),
            pl.BlockSpec((d, h), lambda i: (0, 0)),
            pl.BlockSpec((d, h), lambda i: (0, 0)),
            pl.BlockSpec((1, h), lambda i: (0, 0)),
        ],
        out_specs=[
            pl.BlockSpec((R_BLK, h), lambda i: (i, 0)),
            pl.BlockSpec((R_BLK, h), lambda i: (i, 0)),
        ],
        out_shape=[
            jax.ShapeDtypeStruct((n_pad, h), jnp.float32),
            jax.ShapeDtypeStruct((n_pad, h), jnp.float32),
        ],
    )(xp, w1d, w1b, b1r)

    n_sub = n_pad // SPLIT
    sub_blocks = n_sub // R_BLK
    outs = []
    for s_i in range(SPLIT):
        base_blk = s_i * sub_blocks
        idx_s = pl.pallas_call(
            _make_knn_body(n_real, n_pad, R_BLK, K, N_SEG, base_blk),
            grid=(sub_blocks,),
            scratch_shapes=[
                pltpu.VMEM((R_BLK, n_pad), jnp.float32),
                pltpu.VMEM((K, R_BLK, N_SEG), jnp.float32),
                pltpu.VMEM((K, R_BLK, N_SEG), jnp.int32),
                pltpu.SMEM((1,), jnp.int32),
            ],
            in_specs=[
                pl.BlockSpec((R_BLK, d),
                             lambda i, b=base_blk: (i + b, 0)),
                pl.BlockSpec((d, n_pad), lambda i: (0, 0)),
            ],
            out_specs=pl.BlockSpec((R_BLK, K), lambda i: (i, 0)),
            out_shape=jax.ShapeDtypeStruct((n_sub, K), jnp.int32),
        )(xp, xT)

        idx_flat = idx_s.T.reshape(-1)
        qg = _sc_gather(q, idx_flat, h)
        qg = qg.reshape(K, n_sub, h)

        out_s = pl.pallas_call(
            _make_mlp2_body(K),
            grid=(n_sub // M_BLK,),
            in_specs=[
                pl.BlockSpec((K, M_BLK, h), lambda i: (0, i, 0)),
                pl.BlockSpec((M_BLK, h),
                             lambda i, b=s_i * (n_sub // M_BLK): (i + b, 0)),
                pl.BlockSpec((h, out_dim), lambda i: (0, 0)),
                pl.BlockSpec((1, out_dim), lambda i: (0, 0)),
            ],
            out_specs=pl.BlockSpec((M_BLK, out_dim), lambda i: (i, 0)),
            out_shape=jax.ShapeDtypeStruct((n_sub, out_dim), jnp.float32),
        )(qg, p, W2, b2.reshape(1, out_dim))
        outs.append(out_s)

    out = jnp.concatenate(outs, axis=0)
    return out[:n_real]


# speedup vs baseline: 1.1712x; 1.0016x over previous
import functools

import jax
import jax.numpy as jnp
from jax import lax
from jax.experimental import pallas as pl
from jax.experimental.pallas import tpu as pltpu
from jax.experimental.pallas import tpu_sc as plsc

K = 16
R_BLK = 256
M_BLK = 128
N_SEG = 16
SPLIT = 2


def _make_knn_body(n_real, n_pad, r_blk, k, n_seg, row_blk_off):
    seg_w = n_pad // n_seg

    def body(xb_ref, xT_ref, idx_ref,
             dist_ref, cval_ref, cidx_ref, done_ref):
        i = pl.program_id(0) + row_blk_off
        xb = xb_ref[...]
        xT = xT_ref[...]
        s = lax.dot_general(xb, xT, (((1,), (0,)), ((), ())),
                            preferred_element_type=jnp.float32)
        row_sq = jnp.sum(xb * xb, axis=1, keepdims=True)
        col_sq = jnp.sum(xT * xT, axis=0, keepdims=True)
        dist = row_sq + col_sq - 2.0 * s

        colid = lax.broadcasted_iota(jnp.int32, (1, n_pad), 1)
        rowid = i * r_blk + lax.broadcasted_iota(jnp.int32, (r_blk, 1), 0)
        dist = jnp.where((colid == rowid) | (colid >= n_real), jnp.inf, dist)
        dist_ref[...] = dist

        cval_ref[...] = jnp.full((k, r_blk, n_seg), jnp.inf, jnp.float32)
        cidx_ref[...] = jnp.zeros((k, r_blk, n_seg), jnp.int32)
        done_ref[0] = 0

        colid_seg = lax.broadcasted_iota(jnp.int32, (1, seg_w), 1)

        def round_body(t, carry):
            @pl.when(done_ref[0] == 0)
            def _():
                segs = [dist_ref[:, s * seg_w:(s + 1) * seg_w]
                        for s in range(n_seg)]
                mins = [jnp.min(d, axis=1, keepdims=True) for d in segs]
                m_seg = jnp.concatenate(mins, axis=1)
                floor = jnp.min(m_seg, axis=1, keepdims=True)
                sofar = jnp.concatenate(
                    [cval_ref[t2] for t2 in range(k)], axis=1)
                have = jnp.sum((sofar < floor).astype(jnp.int32), axis=1)
                all_ok = jnp.min(have) >= k

                @pl.when(all_ok)
                def _():
                    done_ref[0] = 1

                @pl.when(jnp.logical_not(all_ok))
                def _():
                    ams = []
                    for s in range(n_seg):
                        am_s = jnp.min(
                            jnp.where(segs[s] == mins[s],
                                      colid_seg, seg_w),
                            axis=1, keepdims=True)
                        dist_ref[:, s * seg_w:(s + 1) * seg_w] = jnp.where(
                            colid_seg == am_s, jnp.inf, segs[s])
                        ams.append(am_s + s * seg_w)
                    cval_ref[t] = m_seg
                    cidx_ref[t] = jnp.concatenate(ams, axis=1)
            return carry

        lax.fori_loop(0, k, round_body, 0)

        vals = jnp.concatenate([cval_ref[t2] for t2 in range(k)], axis=1)
        gidx = jnp.concatenate([cidx_ref[t2] for t2 in range(k)], axis=1)

        idxs = []
        for j in range(k):
            m = jnp.min(vals, axis=1, keepdims=True)
            am = jnp.min(jnp.where(vals == m, gidx, n_pad),
                         axis=1, keepdims=True)
            vals = jnp.where(gidx == am, jnp.inf, vals)
            am = jnp.where(rowid >= n_real,
                           (rowid * k + j) % n_real, am)
            idxs.append(am)
        idx_ref[...] = jnp.concatenate(idxs, axis=1)

    return body


def _make_pq_body():
    def body(xb_ref, w1d_ref, w1b_ref, b1_ref, p_ref, q_ref):
        xb = xb_ref[...]
        p_ref[...] = jnp.dot(xb, w1d_ref[...],
                             preferred_element_type=jnp.float32) + b1_ref[...]
        q_ref[...] = jnp.dot(xb, w1b_ref[...],
                             preferred_element_type=jnp.float32)

    return body


def _make_mlp2_body(k):
    def body(qg_ref, p_ref, w2_ref, b2_ref, o_ref):
        p = p_ref[...]
        w2 = w2_ref[...]
        acc = None
        for j in range(k):
            h = jnp.maximum(p + qg_ref[j], 0.0)
            e = jnp.dot(h, w2, preferred_element_type=jnp.float32)
            acc = e if acc is None else jnp.maximum(acc, e)
        o_ref[...] = acc + b2_ref[...]

    return body


def _sc_gather(table, idx_flat, d):
    b_total = idx_flat.shape[0]
    info = plsc.get_sparse_core_info()
    nw = info.num_cores * info.num_subcores
    b_per_w = b_total // nw
    ch = 128
    n_ch = b_per_w // ch
    mesh = plsc.VectorSubcoreMesh(core_axis_name="c", subcore_axis_name="s")

    @functools.partial(
        pl.kernel, mesh=mesh,
        out_type=jax.ShapeDtypeStruct((b_total, d), jnp.float32),
        scratch_types=[
            pltpu.VMEM((ch,), jnp.int32),
            pltpu.VMEM((ch, d), jnp.float32),
            pltpu.SemaphoreType.DMA,
        ],
    )
    def gk(table_hbm, idx_hbm, out_hbm, idx_v, rows_v, sem):
        wid = lax.axis_index("s") * info.num_cores + lax.axis_index("c")
        base = wid * b_per_w

        def step(c, carry):
            off = base + c * ch
            pltpu.sync_copy(idx_hbm.at[pl.ds(off, ch)], idx_v)
            pltpu.async_copy(table_hbm.at[idx_v], rows_v, sem).wait()
            pltpu.sync_copy(rows_v, out_hbm.at[pl.ds(off, ch)])
            return carry

        lax.fori_loop(0, n_ch, step, 0)

    return gk(table, idx_flat)


def kernel(x, W1, b1, W2, b2):
    n_real, d = x.shape
    h = W1.shape[1]
    out_dim = W2.shape[1]
    quantum = R_BLK * SPLIT
    n_pad = ((n_real + quantum - 1) // quantum) * quantum

    xp = jnp.pad(x, ((0, n_pad - n_real), (0, 0)))
    xT = xp.T
    w1d = W1[:d] - W1[d:]
    w1b = W1[d:]
    b1r = b1.reshape(1, h)

    p, q = pl.pallas_call(
        _make_pq_body(),
        grid=(n_pad // R_BLK,),
        in_specs=[
            pl.BlockSpec((R_BLK, d), lambda i: (i, 0)),
            pl.BlockSpec((d, h), lambda i: (0, 0)),
            pl.BlockSpec((d, h), lambda i: (0, 0)),
            pl.BlockSpec((1, h), lambda i: (0, 0)),
        ],
        out_specs=[
            pl.BlockSpec((R_BLK, h), lambda i: (i, 0)),
            pl.BlockSpec((R_BLK, h), lambda i: (i, 0)),
        ],
        out_shape=[
            jax.ShapeDtypeStruct((n_pad, h), jnp.float32),
            jax.ShapeDtypeStruct((n_pad, h), jnp.float32),
        ],
    )(xp, w1d, w1b, b1r)

    n_sub = n_pad // SPLIT
    sub_blocks = n_sub // R_BLK
    outs = []
    for s_i in range(SPLIT):
        base_blk = s_i * sub_blocks
        idx_s = pl.pallas_call(
            _make_knn_body(n_real, n_pad, R_BLK, K, N_SEG, base_blk),
            grid=(sub_blocks,),
            scratch_shapes=[
                pltpu.VMEM((R_BLK, n_pad), jnp.float32),
                pltpu.VMEM((K, R_BLK, N_SEG), jnp.float32),
                pltpu.VMEM((K, R_BLK, N_SEG), jnp.int32),
                pltpu.SMEM((1,), jnp.int32),
            ],
            in_specs=[
                pl.BlockSpec((R_BLK, d),
                             lambda i, b=base_blk: (i + b, 0)),
                pl.BlockSpec((d, n_pad), lambda i: (0, 0)),
            ],
            out_specs=pl.BlockSpec((R_BLK, K), lambda i: (i, 0)),
            out_shape=jax.ShapeDtypeStruct((n_sub, K), jnp.int32),
        )(xp, xT)

        idx_flat = idx_s.T.reshape(-1)
        qg = _sc_gather(q, idx_flat, h)
        qg = qg.reshape(K, n_sub, h)

        out_s = pl.pallas_call(
            _make_mlp2_body(K),
            grid=(n_sub // M_BLK,),
            in_specs=[
                pl.BlockSpec((K, M_BLK, h), lambda i: (0, i, 0)),
                pl.BlockSpec((M_BLK, h),
                             lambda i, b=s_i * (n_sub // M_BLK): (i + b, 0)),
                pl.BlockSpec((h, out_dim), lambda i: (0, 0)),
                pl.BlockSpec((1, out_dim), lambda i: (0, 0)),
            ],
            out_specs=pl.BlockSpec((M_BLK, out_dim), lambda i: (i, 0)),
            out_shape=jax.ShapeDtypeStruct((n_sub, out_dim), jnp.float32),
        )(qg, p, W2, b2.reshape(1, out_dim))
        outs.append(out_s)

    out = jnp.concatenate(outs, axis=0)
    return out[:n_real]


# speedup vs baseline: 1.1903x; 1.0163x over previous
import functools

import jax
import jax.numpy as jnp
from jax import lax
from jax.experimental import pallas as pl
from jax.experimental.pallas import tpu as pltpu
from jax.experimental.pallas import tpu_sc as plsc

K = 16
R_BLK = 256
M_BLK = 128
N_SEG = 16
SPLIT = 4


def _make_knn_body(n_real, n_pad, r_blk, k, n_seg, row_blk_off):
    seg_w = n_pad // n_seg

    def body(xb_ref, xT_ref, idx_ref,
             dist_ref, cval_ref, cidx_ref, done_ref):
        i = pl.program_id(0) + row_blk_off
        xb = xb_ref[...]
        xT = xT_ref[...]
        s = lax.dot_general(xb, xT, (((1,), (0,)), ((), ())),
                            preferred_element_type=jnp.float32)
        row_sq = jnp.sum(xb * xb, axis=1, keepdims=True)
        col_sq = jnp.sum(xT * xT, axis=0, keepdims=True)
        dist = row_sq + col_sq - 2.0 * s

        colid = lax.broadcasted_iota(jnp.int32, (1, n_pad), 1)
        rowid = i * r_blk + lax.broadcasted_iota(jnp.int32, (r_blk, 1), 0)
        dist = jnp.where((colid == rowid) | (colid >= n_real), jnp.inf, dist)
        dist_ref[...] = dist

        cval_ref[...] = jnp.full((k, r_blk, n_seg), jnp.inf, jnp.float32)
        cidx_ref[...] = jnp.zeros((k, r_blk, n_seg), jnp.int32)
        done_ref[0] = 0

        colid_seg = lax.broadcasted_iota(jnp.int32, (1, seg_w), 1)

        def round_body(t, carry):
            @pl.when(done_ref[0] == 0)
            def _():
                segs = [dist_ref[:, s * seg_w:(s + 1) * seg_w]
                        for s in range(n_seg)]
                mins = [jnp.min(d, axis=1, keepdims=True) for d in segs]
                m_seg = jnp.concatenate(mins, axis=1)
                floor = jnp.min(m_seg, axis=1, keepdims=True)
                sofar = jnp.concatenate(
                    [cval_ref[t2] for t2 in range(k)], axis=1)
                have = jnp.sum((sofar < floor).astype(jnp.int32), axis=1)
                all_ok = jnp.min(have) >= k

                @pl.when(all_ok)
                def _():
                    done_ref[0] = 1

                @pl.when(jnp.logical_not(all_ok))
                def _():
                    ams = []
                    for s in range(n_seg):
                        am_s = jnp.min(
                            jnp.where(segs[s] == mins[s],
                                      colid_seg, seg_w),
                            axis=1, keepdims=True)
                        dist_ref[:, s * seg_w:(s + 1) * seg_w] = jnp.where(
                            colid_seg == am_s, jnp.inf, segs[s])
                        ams.append(am_s + s * seg_w)
                    cval_ref[t] = m_seg
                    cidx_ref[t] = jnp.concatenate(ams, axis=1)
            return carry

        lax.fori_loop(0, k, round_body, 0)

        vals = jnp.concatenate([cval_ref[t2] for t2 in range(k)], axis=1)
        gidx = jnp.concatenate([cidx_ref[t2] for t2 in range(k)], axis=1)

        idxs = []
        for j in range(k):
            m = jnp.min(vals, axis=1, keepdims=True)
            am = jnp.min(jnp.where(vals == m, gidx, n_pad),
                         axis=1, keepdims=True)
            vals = jnp.where(gidx == am, jnp.inf, vals)
            am = jnp.where(rowid >= n_real,
                           (rowid * k + j) % n_real, am)
            idxs.append(am)
        idx_ref[...] = jnp.concatenate(idxs, axis=1)

    return body


def _make_pq_body():
    def body(xb_ref, w1d_ref, w1b_ref, b1_ref, p_ref, q_ref):
        xb = xb_ref[...]
        p_ref[...] = jnp.dot(xb, w1d_ref[...],
                             preferred_element_type=jnp.float32) + b1_ref[...]
        q_ref[...] = jnp.dot(xb, w1b_ref[...],
                             preferred_element_type=jnp.float32)

    return body


def _make_mlp2_body(k):
    def body(qg_ref, p_ref, w2_ref, b2_ref, o_ref):
        p = p_ref[...]
        w2 = w2_ref[...]
        acc = None
        for j in range(k):
            h = jnp.maximum(p + qg_ref[j], 0.0)
            e = jnp.dot(h, w2, preferred_element_type=jnp.float32)
            acc = e if acc is None else jnp.maximum(acc, e)
        o_ref[...] = acc + b2_ref[...]

    return body


def _sc_gather(table, idx_flat, d):
    b_total = idx_flat.shape[0]
    info = plsc.get_sparse_core_info()
    nw = info.num_cores * info.num_subcores
    b_per_w = b_total // nw
    ch = 128
    n_ch = b_per_w // ch
    mesh = plsc.VectorSubcoreMesh(core_axis_name="c", subcore_axis_name="s")

    @functools.partial(
        pl.kernel, mesh=mesh,
        out_type=jax.ShapeDtypeStruct((b_total, d), jnp.float32),
        scratch_types=[
            pltpu.VMEM((ch,), jnp.int32),
            pltpu.VMEM((ch, d), jnp.float32),
            pltpu.SemaphoreType.DMA,
        ],
    )
    def gk(table_hbm, idx_hbm, out_hbm, idx_v, rows_v, sem):
        wid = lax.axis_index("s") * info.num_cores + lax.axis_index("c")
        base = wid * b_per_w

        def step(c, carry):
            off = base + c * ch
            pltpu.sync_copy(idx_hbm.at[pl.ds(off, ch)], idx_v)
            pltpu.async_copy(table_hbm.at[idx_v], rows_v, sem).wait()
            pltpu.sync_copy(rows_v, out_hbm.at[pl.ds(off, ch)])
            return carry

        lax.fori_loop(0, n_ch, step, 0)

    return gk(table, idx_flat)


def kernel(x, W1, b1, W2, b2):
    n_real, d = x.shape
    h = W1.shape[1]
    out_dim = W2.shape[1]
    quantum = R_BLK * SPLIT
    n_pad = ((n_real + quantum - 1) // quantum) * quantum

    xp = jnp.pad(x, ((0, n_pad - n_real), (0, 0)))
    xT = xp.T
    w1d = W1[:d] - W1[d:]
    w1b = W1[d:]
    b1r = b1.reshape(1, h)

    p, q = pl.pallas_call(
        _make_pq_body(),
        grid=(n_pad // R_BLK,),
        in_specs=[
            pl.BlockSpec((R_BLK, d), lambda i: (i, 0)),
            pl.BlockSpec((d, h), lambda i: (0, 0)),
            pl.BlockSpec((d, h), lambda i: (0, 0)),
            pl.BlockSpec((1, h), lambda i: (0, 0)),
        ],
        out_specs=[
            pl.BlockSpec((R_BLK, h), lambda i: (i, 0)),
            pl.BlockSpec((R_BLK, h), lambda i: (i, 0)),
        ],
        out_shape=[
            jax.ShapeDtypeStruct((n_pad, h), jnp.float32),
            jax.ShapeDtypeStruct((n_pad, h), jnp.float32),
        ],
    )(xp, w1d, w1b, b1r)

    n_sub = n_pad // SPLIT
    sub_blocks = n_sub // R_BLK
    outs = []
    for s_i in range(SPLIT):
        base_blk = s_i * sub_blocks
        idx_s = pl.pallas_call(
            _make_knn_body(n_real, n_pad, R_BLK, K, N_SEG, base_blk),
            grid=(sub_blocks,),
            scratch_shapes=[
                pltpu.VMEM((R_BLK, n_pad), jnp.float32),
                pltpu.VMEM((K, R_BLK, N_SEG), jnp.float32),
                pltpu.VMEM((K, R_BLK, N_SEG), jnp.int32),
                pltpu.SMEM((1,), jnp.int32),
            ],
            in_specs=[
                pl.BlockSpec((R_BLK, d),
                             lambda i, b=base_blk: (i + b, 0)),
                pl.BlockSpec((d, n_pad), lambda i: (0, 0)),
            ],
            out_specs=pl.BlockSpec((R_BLK, K), lambda i: (i, 0)),
            out_shape=jax.ShapeDtypeStruct((n_sub, K), jnp.int32),
        )(xp, xT)

        idx_flat = idx_s.T.reshape(-1)
        qg = _sc_gather(q, idx_flat, h)
        qg = qg.reshape(K, n_sub, h)

        out_s = pl.pallas_call(
            _make_mlp2_body(K),
            grid=(n_sub // M_BLK,),
            in_specs=[
                pl.BlockSpec((K, M_BLK, h), lambda i: (0, i, 0)),
                pl.BlockSpec((M_BLK, h),
                             lambda i, b=s_i * (n_sub // M_BLK): (i + b, 0)),
                pl.BlockSpec((h, out_dim), lambda i: (0, 0)),
                pl.BlockSpec((1, out_dim), lambda i: (0, 0)),
            ],
            out_specs=pl.BlockSpec((M_BLK, out_dim), lambda i: (i, 0)),
            out_shape=jax.ShapeDtypeStruct((n_sub, out_dim), jnp.float32),
        )(qg, p, W2, b2.reshape(1, out_dim))
        outs.append(out_s)

    out = jnp.concatenate(outs, axis=0)
    return out[:n_real]
